# Initial kernel scaffold; baseline (speedup 1.0000x reference)
import jax
import jax.numpy as jnp
from jax.experimental import pallas as pl


def kernel(x, memory, read_w, write_w, hidden, w_ih, w_hh, b_ih, b_hh, proj_w, proj_b):
    raise NotImplementedError("write your pallas kernel here")


# speedup vs baseline: 1.8691x; 1.8691x over previous
import functools

import jax
import jax.numpy as jnp
import numpy as np
from jax.experimental import pallas as pl
from jax.experimental.pallas import tpu as pltpu

_EPS = 1e-12
_BB = 8


def _address(cos, beta, gate, shift, gamma, prev_w):
    b = jax.nn.softplus(beta)
    a = b * cos
    a = a - jnp.max(a, axis=-1, keepdims=True)
    e = jnp.exp(a)
    wt = e / jnp.sum(e, axis=-1, keepdims=True)
    g = jax.nn.sigmoid(gate)
    wt = g * wt + (1.0 - g) * prev_w
    s = shift - jnp.max(shift, axis=-1, keepdims=True)
    s = jnp.exp(s)
    s = s / jnp.sum(s, axis=-1, keepdims=True)
    wm1 = jnp.concatenate([wt[:, 1:], wt[:, :1]], axis=1)
    wp1 = jnp.concatenate([wt[:, -1:], wt[:, :-1]], axis=1)
    wt = s[:, 0:1] * wm1 + s[:, 1:2] * wt + s[:, 2:3] * wp1
    gam = 1.0 + jax.nn.softplus(gamma)
    wt = jnp.exp(gam * jnp.log(wt + _EPS))
    return wt / (jnp.sum(wt, axis=-1, keepdims=True) + _EPS)


def _ntm_body(x_ref, mem_ref, rw_ref, ww_ref, h_ref, wih_ref, whh_ref,
              bih_ref, bhh_ref, pw_ref, pb_ref,
              y_ref, nm_ref, nrw_ref, wtw_ref, *, H):
    M = x_ref.shape[1]
    mem = mem_ref[...]
    rw = rw_ref[...]
    ww = ww_ref[...]

    read_vec = jnp.sum(rw[:, :, None] * mem, axis=1)

    h = h_ref[...]
    wih = wih_ref[...]
    whh = whh_ref[...]
    bih = bih_ref[...]
    bhh = bhh_ref[...]

    def gru(xt, h):
        gi = jnp.dot(xt, wih) + bih
        gh = jnp.dot(h, whh) + bhh
        r = jax.nn.sigmoid(gi[:, :H] + gh[:, :H])
        z = jax.nn.sigmoid(gi[:, H:2 * H] + gh[:, H:2 * H])
        n = jnp.tanh(gi[:, 2 * H:] + r * gh[:, 2 * H:])
        return (1.0 - z) * n + z * h

    h = gru(x_ref[...], h)
    h = gru(read_vec, h)
    co = jnp.dot(h, pw_ref[...]) + pb_ref[...]

    key_r = co[:, :M]
    key_w = co[:, M:2 * M]
    erase = jax.nn.sigmoid(co[:, 2 * M:3 * M])
    addv = co[:, 3 * M:4 * M]
    y_ref[...] = co[:, 4 * M:5 * M]
    sc = co[:, 5 * M:5 * M + 12]

    inv_norm = 1.0 / (jnp.sqrt(jnp.sum(mem * mem, axis=2)) + _EPS)
    kr = key_r / (jnp.sqrt(jnp.sum(key_r * key_r, axis=1, keepdims=True)) + _EPS)
    kw = key_w / (jnp.sqrt(jnp.sum(key_w * key_w, axis=1, keepdims=True)) + _EPS)
    cos_r = jnp.sum(mem * kr[:, None, :], axis=2) * inv_norm
    cos_w = jnp.sum(mem * kw[:, None, :], axis=2) * inv_norm

    new_rw = _address(cos_r, sc[:, 0:1], sc[:, 1:2], sc[:, 2:5], sc[:, 5:6], rw)
    wt_w = _address(cos_w, sc[:, 6:7], sc[:, 7:8], sc[:, 8:11], sc[:, 11:12], ww)
    nrw_ref[...] = new_rw
    wtw_ref[...] = wt_w

    nm_ref[...] = (mem * (1.0 - wt_w[:, :, None] * erase[:, None, :])
                   + wt_w[:, :, None] * addv[:, None, :])


def kernel(x, memory, read_w, write_w, hidden, w_ih, w_hh, b_ih, b_hh,
           proj_w, proj_b):
    B, N, M = memory.shape
    H = hidden.shape[-1]
    rl = M + 6
    perm = np.concatenate([
        np.arange(0, M),
        np.arange(rl, rl + M),
        np.arange(rl + M + 6, rl + 2 * M + 6),
        np.arange(rl + 2 * M + 6, rl + 3 * M + 6),
        np.arange(rl + 3 * M + 6, rl + 3 * M + 6 + M),
        np.arange(M, M + 6),
        np.arange(rl + M, rl + M + 6),
    ])
    P = 5 * M + 12
    PP = ((P + 127) // 128) * 128
    pw = jnp.zeros((H, PP), jnp.float32).at[:, :P].set(proj_w[perm].T)
    pb = jnp.zeros((1, PP), jnp.float32).at[:, :P].set(proj_b[perm][None])

    grid = (B // _BB,)
    body = functools.partial(_ntm_body, H=H)
    y, new_mem, nrw, wtw = pl.pallas_call(
        body,
        grid=grid,
        in_specs=[
            pl.BlockSpec((_BB, M), lambda i: (i, 0)),
            pl.BlockSpec((_BB, N, M), lambda i: (i, 0, 0)),
            pl.BlockSpec((_BB, N), lambda i: (i, 0)),
            pl.BlockSpec((_BB, N), lambda i: (i, 0)),
            pl.BlockSpec((_BB, H), lambda i: (i, 0)),
            pl.BlockSpec((M, 3 * H), lambda i: (0, 0)),
            pl.BlockSpec((H, 3 * H), lambda i: (0, 0)),
            pl.BlockSpec((1, 3 * H), lambda i: (0, 0)),
            pl.BlockSpec((1, 3 * H), lambda i: (0, 0)),
            pl.BlockSpec((H, PP), lambda i: (0, 0)),
            pl.BlockSpec((1, PP), lambda i: (0, 0)),
        ],
        out_specs=[
            pl.BlockSpec((_BB, M), lambda i: (i, 0)),
            pl.BlockSpec((_BB, N, M), lambda i: (i, 0, 0)),
            pl.BlockSpec((_BB, N), lambda i: (i, 0)),
            pl.BlockSpec((_BB, N), lambda i: (i, 0)),
        ],
        out_shape=[
            jax.ShapeDtypeStruct((B, M), jnp.float32),
            jax.ShapeDtypeStruct((B, N, M), jnp.float32),
            jax.ShapeDtypeStruct((B, N), jnp.float32),
            jax.ShapeDtypeStruct((B, N), jnp.float32),
        ],
        compiler_params=pltpu.CompilerParams(
            dimension_semantics=("parallel",),
            vmem_limit_bytes=56 * 1024 * 1024,
        ),
    )(x, memory, read_w[:, 0, :], write_w[:, 0, :], hidden[0],
      w_ih.T, w_hh.T, b_ih[None], b_hh[None], pw, pb)

    return y, new_mem, nrw[:, None, :], wtw[:, None, :]


# speedup vs baseline: 2.3575x; 1.2613x over previous
import functools

import jax
import jax.numpy as jnp
import numpy as np
from jax.experimental import pallas as pl
from jax.experimental.pallas import tpu as pltpu

_EPS = 1e-12
_BB = 8


def _address(cos, beta, gate, shift, gamma, prev_w):
    b = jax.nn.softplus(beta)
    a = b * cos
    a = a - jnp.max(a, axis=-1, keepdims=True)
    e = jnp.exp(a)
    wt = e / jnp.sum(e, axis=-1, keepdims=True)
    g = jax.nn.sigmoid(gate)
    wt = g * wt + (1.0 - g) * prev_w
    s = shift - jnp.max(shift, axis=-1, keepdims=True)
    s = jnp.exp(s)
    s = s / jnp.sum(s, axis=-1, keepdims=True)
    wm1 = jnp.concatenate([wt[:, 1:], wt[:, :1]], axis=1)
    wp1 = jnp.concatenate([wt[:, -1:], wt[:, :-1]], axis=1)
    wt = s[:, 0:1] * wm1 + s[:, 1:2] * wt + s[:, 2:3] * wp1
    gam = 1.0 + jax.nn.softplus(gamma)
    wt = jnp.exp(gam * jnp.log(wt + _EPS))
    return wt / (jnp.sum(wt, axis=-1, keepdims=True) + _EPS)


def _ntm_body(x_ref, mem_ref, rw_ref, ww_ref, h_ref, wih_ref, whh_ref,
              bih_ref, bhh_ref, pw_ref, pb_ref,
              y_ref, nm_ref, nrw_ref, wtw_ref, *, H):
    M = x_ref.shape[1]
    Bb = x_ref.shape[0]
    mem = mem_ref[...]
    rw = rw_ref[...]
    ww = ww_ref[...]

    read_vec = jnp.concatenate(
        [jnp.dot(rw[b:b + 1, :], mem[b]) for b in range(Bb)], axis=0)

    h = h_ref[...]
    wih = wih_ref[...]
    whh = whh_ref[...]
    bih = bih_ref[...]
    bhh = bhh_ref[...]

    def gru(xt, h):
        gi = jnp.dot(xt, wih) + bih
        gh = jnp.dot(h, whh) + bhh
        r = jax.nn.sigmoid(gi[:, :H] + gh[:, :H])
        z = jax.nn.sigmoid(gi[:, H:2 * H] + gh[:, H:2 * H])
        n = jnp.tanh(gi[:, 2 * H:] + r * gh[:, 2 * H:])
        return (1.0 - z) * n + z * h

    h = gru(x_ref[...], h)
    h = gru(read_vec, h)
    co = jnp.dot(h, pw_ref[...]) + pb_ref[...]

    key_r = co[:, :M]
    key_w = co[:, M:2 * M]
    erase = jax.nn.sigmoid(co[:, 2 * M:3 * M])
    addv = co[:, 3 * M:4 * M]
    y_ref[...] = co[:, 4 * M:5 * M]
    sc = co[:, 5 * M:5 * M + 12]

    inv_norm = 1.0 / (jnp.sqrt(jnp.sum(mem * mem, axis=2)) + _EPS)
    kr = key_r / (jnp.sqrt(jnp.sum(key_r * key_r, axis=1, keepdims=True)) + _EPS)
    kw = key_w / (jnp.sqrt(jnp.sum(key_w * key_w, axis=1, keepdims=True)) + _EPS)

    dcols = []
    for b in range(Bb):
        keys2 = jnp.concatenate([kr[b:b + 1, :], kw[b:b + 1, :]], axis=0)
        dcols.append(jax.lax.dot_general(
            mem[b], keys2, (((1,), (1,)), ((), ()))))
    d3 = jnp.swapaxes(jnp.stack(dcols, axis=0), 1, 2)
    cos_r = d3[:, 0, :] * inv_norm
    cos_w = d3[:, 1, :] * inv_norm

    new_rw = _address(cos_r, sc[:, 0:1], sc[:, 1:2], sc[:, 2:5], sc[:, 5:6], rw)
    wt_w = _address(cos_w, sc[:, 6:7], sc[:, 7:8], sc[:, 8:11], sc[:, 11:12], ww)
    nrw_ref[...] = new_rw
    wtw_ref[...] = wt_w

    nm_ref[...] = mem + wt_w[:, :, None] * (addv[:, None, :]
                                            - erase[:, None, :] * mem)


def kernel(x, memory, read_w, write_w, hidden, w_ih, w_hh, b_ih, b_hh,
           proj_w, proj_b):
    B, N, M = memory.shape
    H = hidden.shape[-1]
    rl = M + 6
    perm = np.concatenate([
        np.arange(0, M),
        np.arange(rl, rl + M),
        np.arange(rl + M + 6, rl + 2 * M + 6),
        np.arange(rl + 2 * M + 6, rl + 3 * M + 6),
        np.arange(rl + 3 * M + 6, rl + 3 * M + 6 + M),
        np.arange(M, M + 6),
        np.arange(rl + M, rl + M + 6),
    ])
    P = 5 * M + 12
    PP = ((P + 127) // 128) * 128
    pw = jnp.zeros((H, PP), jnp.float32).at[:, :P].set(proj_w[perm].T)
    pb = jnp.zeros((1, PP), jnp.float32).at[:, :P].set(proj_b[perm][None])

    grid = (B // _BB,)
    body = functools.partial(_ntm_body, H=H)
    y, new_mem, nrw, wtw = pl.pallas_call(
        body,
        grid=grid,
        in_specs=[
            pl.BlockSpec((_BB, M), lambda i: (i, 0)),
            pl.BlockSpec((_BB, N, M), lambda i: (i, 0, 0)),
            pl.BlockSpec((_BB, N), lambda i: (i, 0)),
            pl.BlockSpec((_BB, N), lambda i: (i, 0)),
            pl.BlockSpec((_BB, H), lambda i: (i, 0)),
            pl.BlockSpec((M, 3 * H), lambda i: (0, 0)),
            pl.BlockSpec((H, 3 * H), lambda i: (0, 0)),
            pl.BlockSpec((1, 3 * H), lambda i: (0, 0)),
            pl.BlockSpec((1, 3 * H), lambda i: (0, 0)),
            pl.BlockSpec((H, PP), lambda i: (0, 0)),
            pl.BlockSpec((1, PP), lambda i: (0, 0)),
        ],
        out_specs=[
            pl.BlockSpec((_BB, M), lambda i: (i, 0)),
            pl.BlockSpec((_BB, N, M), lambda i: (i, 0, 0)),
            pl.BlockSpec((_BB, N), lambda i: (i, 0)),
            pl.BlockSpec((_BB, N), lambda i: (i, 0)),
        ],
        out_shape=[
            jax.ShapeDtypeStruct((B, M), jnp.float32),
            jax.ShapeDtypeStruct((B, N, M), jnp.float32),
            jax.ShapeDtypeStruct((B, N), jnp.float32),
            jax.ShapeDtypeStruct((B, N), jnp.float32),
        ],
        compiler_params=pltpu.CompilerParams(
            dimension_semantics=("parallel",),
            vmem_limit_bytes=56 * 1024 * 1024,
        ),
    )(x, memory, read_w[:, 0, :], write_w[:, 0, :], hidden[0],
      w_ih.T, w_hh.T, b_ih[None], b_hh[None], pw, pb)

    return y, new_mem, nrw[:, None, :], wtw[:, None, :]
